# Initial kernel scaffold; baseline (speedup 1.0000x reference)
import jax
import jax.numpy as jnp
from jax.experimental import pallas as pl


def kernel(x, t, W, b):
    raise NotImplementedError("write your pallas kernel here")


# speedup vs baseline: 5.3727x; 5.3727x over previous
import jax
import jax.numpy as jnp
from jax import lax
from jax.experimental import pallas as pl

N = 8192
D = 768
MT = 16
MY = 32
BN = 512


def _body(x_ref, t_ref, w_ref, b_ref, o_ref):
    xb = x_ref[...]
    logits = jnp.dot(xb, w_ref[...], preferred_element_type=jnp.float32)
    logits = logits + b_ref[...]
    tb = t_ref[...]
    col_task = lax.broadcasted_iota(jnp.int32, (BN, MT * MY), 1) // MY
    sel = col_task == tb
    masked = jnp.where(sel, logits, -1e30)
    m = jnp.max(masked, axis=1, keepdims=True)
    p = jnp.exp(masked - m)
    p = p / jnp.sum(p, axis=1, keepdims=True)
    comp = (lax.broadcasted_iota(jnp.int32, (MT * MY, MY), 0) % MY ==
            lax.broadcasted_iota(jnp.int32, (MT * MY, MY), 1))
    o_ref[...] = jnp.dot(p, comp.astype(jnp.float32),
                         preferred_element_type=jnp.float32)


def kernel(x, t, W, b):
    w2 = W.reshape(MT * MY, D).T
    b2 = b.reshape(1, MT * MY)
    t2 = t.reshape(N, 1)
    return pl.pallas_call(
        _body,
        grid=(N // BN,),
        in_specs=[
            pl.BlockSpec((BN, D), lambda i: (i, 0)),
            pl.BlockSpec((BN, 1), lambda i: (i, 0)),
            pl.BlockSpec((D, MT * MY), lambda i: (0, 0)),
            pl.BlockSpec((1, MT * MY), lambda i: (0, 0)),
        ],
        out_specs=pl.BlockSpec((BN, MY), lambda i: (i, 0)),
        out_shape=jax.ShapeDtypeStruct((N, MY), jnp.float32),
    )(x, t2, w2, b2)


# speedup vs baseline: 5.6763x; 1.0565x over previous
import jax
import jax.numpy as jnp
from jax import lax
from jax.experimental import pallas as pl

N = 8192
D = 768
MT = 16
MY = 32
BN = 512


def _body(x_ref, t_ref, w_ref, b_ref, o_ref):
    xb = x_ref[...]
    logits = jnp.dot(xb, w_ref[...], preferred_element_type=jnp.float32)
    logits = logits + b_ref[...]
    tb = t_ref[...]
    col_task = lax.broadcasted_iota(jnp.int32, (BN, MT * MY), 1) // MY
    sel = col_task == tb
    masked = jnp.where(sel, logits, 0.0)
    comp = (lax.broadcasted_iota(jnp.int32, (MT * MY, MY), 0) % MY ==
            lax.broadcasted_iota(jnp.int32, (MT * MY, MY), 1))
    zl = jnp.dot(masked, comp.astype(jnp.float32),
                 preferred_element_type=jnp.float32)
    m = jnp.max(zl, axis=1, keepdims=True)
    p = jnp.exp(zl - m)
    o_ref[...] = p / jnp.sum(p, axis=1, keepdims=True)


def kernel(x, t, W, b):
    w2 = W.reshape(MT * MY, D).T
    b2 = b.reshape(1, MT * MY)
    t2 = t.reshape(N, 1)
    return pl.pallas_call(
        _body,
        grid=(N // BN,),
        in_specs=[
            pl.BlockSpec((BN, D), lambda i: (i, 0)),
            pl.BlockSpec((BN, 1), lambda i: (i, 0)),
            pl.BlockSpec((D, MT * MY), lambda i: (0, 0)),
            pl.BlockSpec((1, MT * MY), lambda i: (0, 0)),
        ],
        out_specs=pl.BlockSpec((BN, MY), lambda i: (i, 0)),
        out_shape=jax.ShapeDtypeStruct((N, MY), jnp.float32),
    )(x, t2, w2, b2)


# speedup vs baseline: 5.7673x; 1.0160x over previous
import jax
import jax.numpy as jnp
from jax import lax
from jax.experimental import pallas as pl

N = 8192
D = 768
MT = 16
MY = 32
BN = 512


def _body(x_ref, t_ref, w_ref, b_ref, o_ref):
    xb = x_ref[...].astype(jnp.bfloat16)
    logits = jnp.dot(xb, w_ref[...], preferred_element_type=jnp.float32)
    logits = logits + b_ref[...]
    tb = t_ref[...]
    col_task = lax.broadcasted_iota(jnp.int32, (BN, MT * MY), 1) // MY
    sel = col_task == tb
    masked = jnp.where(sel, logits, 0.0)
    comp = (lax.broadcasted_iota(jnp.int32, (MT * MY, MY), 0) % MY ==
            lax.broadcasted_iota(jnp.int32, (MT * MY, MY), 1))
    zl = jnp.dot(masked, comp.astype(jnp.float32),
                 preferred_element_type=jnp.float32)
    m = jnp.max(zl, axis=1, keepdims=True)
    p = jnp.exp(zl - m)
    o_ref[...] = p / jnp.sum(p, axis=1, keepdims=True)


def kernel(x, t, W, b):
    w2 = W.reshape(MT * MY, D).T.astype(jnp.bfloat16)
    b2 = b.reshape(1, MT * MY)
    t2 = t.reshape(N, 1)
    return pl.pallas_call(
        _body,
        grid=(N // BN,),
        in_specs=[
            pl.BlockSpec((BN, D), lambda i: (i, 0)),
            pl.BlockSpec((BN, 1), lambda i: (i, 0)),
            pl.BlockSpec((D, MT * MY), lambda i: (0, 0)),
            pl.BlockSpec((1, MT * MY), lambda i: (0, 0)),
        ],
        out_specs=pl.BlockSpec((BN, MY), lambda i: (i, 0)),
        out_shape=jax.ShapeDtypeStruct((N, MY), jnp.float32),
    )(x, t2, w2, b2)


# speedup vs baseline: 6.9021x; 1.1968x over previous
import jax
import jax.numpy as jnp
from jax import lax
from jax.experimental import pallas as pl

N = 8192
D = 768
MT = 16
MY = 32
BN = 1024


def _body(x_ref, t_ref, w_ref, b_ref, o_ref):
    xb = x_ref[...].astype(jnp.bfloat16)
    logits = jnp.dot(xb, w_ref[...], preferred_element_type=jnp.float32)
    logits = logits + b_ref[...]
    tb = t_ref[...]
    col_task = lax.broadcasted_iota(jnp.int32, (BN, MT * MY), 1) // MY
    sel = col_task == tb
    masked = jnp.where(sel, logits, 0.0)
    comp = (lax.broadcasted_iota(jnp.int32, (MT * MY, MY), 0) % MY ==
            lax.broadcasted_iota(jnp.int32, (MT * MY, MY), 1))
    zl = jnp.dot(masked, comp.astype(jnp.float32),
                 preferred_element_type=jnp.float32)
    m = jnp.max(zl, axis=1, keepdims=True)
    p = jnp.exp(zl - m)
    o_ref[...] = p / jnp.sum(p, axis=1, keepdims=True)


def kernel(x, t, W, b):
    w2 = W.reshape(MT * MY, D).T.astype(jnp.bfloat16)
    b2 = b.reshape(1, MT * MY)
    t2 = t.reshape(N, 1)
    return pl.pallas_call(
        _body,
        grid=(N // BN,),
        in_specs=[
            pl.BlockSpec((BN, D), lambda i: (i, 0)),
            pl.BlockSpec((BN, 1), lambda i: (i, 0)),
            pl.BlockSpec((D, MT * MY), lambda i: (0, 0)),
            pl.BlockSpec((1, MT * MY), lambda i: (0, 0)),
        ],
        out_specs=pl.BlockSpec((BN, MY), lambda i: (i, 0)),
        out_shape=jax.ShapeDtypeStruct((N, MY), jnp.float32),
    )(x, t2, w2, b2)


# speedup vs baseline: 7.4076x; 1.0732x over previous
import jax
import jax.numpy as jnp
from jax import lax
from jax.experimental import pallas as pl

N = 8192
D = 768
MT = 16
MY = 32
BN = 2048


def _body(x_ref, t_ref, w_ref, b_ref, o_ref):
    xb = x_ref[...].astype(jnp.bfloat16)
    logits = jnp.dot(xb, w_ref[...], preferred_element_type=jnp.float32)
    logits = logits + b_ref[...]
    tb = t_ref[...]
    col_task = lax.broadcasted_iota(jnp.int32, (BN, MT * MY), 1) // MY
    sel = col_task == tb
    masked = jnp.where(sel, logits, 0.0)
    comp = (lax.broadcasted_iota(jnp.int32, (MT * MY, MY), 0) % MY ==
            lax.broadcasted_iota(jnp.int32, (MT * MY, MY), 1))
    zl = jnp.dot(masked, comp.astype(jnp.float32),
                 preferred_element_type=jnp.float32)
    m = jnp.max(zl, axis=1, keepdims=True)
    p = jnp.exp(zl - m)
    o_ref[...] = p / jnp.sum(p, axis=1, keepdims=True)


def kernel(x, t, W, b):
    w2 = W.reshape(MT * MY, D).T.astype(jnp.bfloat16)
    b2 = b.reshape(1, MT * MY)
    t2 = t.reshape(N, 1)
    return pl.pallas_call(
        _body,
        grid=(N // BN,),
        in_specs=[
            pl.BlockSpec((BN, D), lambda i: (i, 0)),
            pl.BlockSpec((BN, 1), lambda i: (i, 0)),
            pl.BlockSpec((D, MT * MY), lambda i: (0, 0)),
            pl.BlockSpec((1, MT * MY), lambda i: (0, 0)),
        ],
        out_specs=pl.BlockSpec((BN, MY), lambda i: (i, 0)),
        out_shape=jax.ShapeDtypeStruct((N, MY), jnp.float32),
    )(x, t2, w2, b2)
